# baseline (speedup 1.0000x reference)
import functools

import jax
import jax.numpy as jnp
from jax import lax
from jax.experimental import pallas as pl
from jax.experimental.pallas import tpu as pltpu
from jax.experimental.pallas import tpu_sc as plsc

NUM_CORES = 2
NUM_SUBCORES = 16
NW = NUM_CORES * NUM_SUBCORES
CHUNK = 128


def _sc_gather(table_flat, idx_flat):
    total, dim = idx_flat.shape[0], table_flat.shape[1]
    per_w = total // NW
    n_chunks = per_w // CHUNK
    mesh = plsc.VectorSubcoreMesh(core_axis_name="c", subcore_axis_name="s")

    @functools.partial(
        pl.kernel,
        mesh=mesh,
        compiler_params=pltpu.CompilerParams(use_tc_tiling_on_sc=False),
        out_type=jax.ShapeDtypeStruct((total, dim), jnp.float32),
        scratch_types=[
            pltpu.VMEM((per_w,), jnp.int32),
            pltpu.VMEM((CHUNK, dim), jnp.float32),
            pltpu.VMEM((CHUNK, dim), jnp.float32),
            pltpu.SemaphoreType.DMA,
            pltpu.SemaphoreType.DMA,
        ],
    )
    def k(table_hbm, idx_hbm, out_hbm, idx_v, buf0, buf1, sem0, sem1):
        wid = lax.axis_index("s") * NUM_CORES + lax.axis_index("c")
        base = wid * per_w
        pltpu.sync_copy(idx_hbm.at[pl.ds(base, per_w)], idx_v)

        pltpu.async_copy(table_hbm.at[idx_v.at[pl.ds(0, CHUNK)]], buf0, sem0)
        pltpu.async_copy(table_hbm.at[idx_v.at[pl.ds(CHUNK, CHUNK)]], buf1, sem1)

        def body(j, carry):
            c0 = 2 * j
            pltpu.make_async_copy(
                table_hbm.at[idx_v.at[pl.ds(c0 * CHUNK, CHUNK)]], buf0, sem0
            ).wait()
            pltpu.sync_copy(buf0, out_hbm.at[pl.ds(base + c0 * CHUNK, CHUNK)])

            @pl.when(c0 + 2 < n_chunks)
            def _():
                pltpu.async_copy(
                    table_hbm.at[idx_v.at[pl.ds((c0 + 2) * CHUNK, CHUNK)]],
                    buf0,
                    sem0,
                )

            c1 = c0 + 1
            pltpu.make_async_copy(
                table_hbm.at[idx_v.at[pl.ds(c1 * CHUNK, CHUNK)]], buf1, sem1
            ).wait()
            pltpu.sync_copy(buf1, out_hbm.at[pl.ds(base + c1 * CHUNK, CHUNK)])

            @pl.when(c1 + 2 < n_chunks)
            def _():
                pltpu.async_copy(
                    table_hbm.at[idx_v.at[pl.ds((c1 + 2) * CHUNK, CHUNK)]],
                    buf1,
                    sem1,
                )

            return carry

        lax.fori_loop(0, n_chunks // 2, body, 0)

    return k(table_flat, idx_flat)


def kernel(inputs, tables):
    fields, vocab, dim = tables.shape
    batch = inputs.shape[0]
    offsets = (jnp.arange(fields, dtype=jnp.int32) * vocab)[None, :]
    idx_flat = (inputs + offsets).reshape(batch * fields)
    table_flat = tables.reshape(fields * vocab, dim)
    out = _sc_gather(table_flat, idx_flat)
    return out.reshape(batch, fields, dim)


# speedup vs baseline: 1.1081x; 1.1081x over previous
import functools

import jax
import jax.numpy as jnp
from jax import lax
from jax.experimental import pallas as pl
from jax.experimental.pallas import tpu as pltpu
from jax.experimental.pallas import tpu_sc as plsc

NUM_CORES = 2
NUM_SUBCORES = 16
NW = NUM_CORES * NUM_SUBCORES
CHUNK = 128
PDIM = 128


def _sc_gather(table_flat, idx_flat):
    total = idx_flat.shape[0]
    per_w = total // NW
    n_chunks = per_w // CHUNK
    mesh = plsc.VectorSubcoreMesh(core_axis_name="c", subcore_axis_name="s")

    @functools.partial(
        pl.kernel,
        mesh=mesh,
        compiler_params=pltpu.CompilerParams(use_tc_tiling_on_sc=True),
        out_type=jax.ShapeDtypeStruct((total, PDIM), jnp.float32),
        scratch_types=[
            pltpu.VMEM((per_w,), jnp.int32),
            pltpu.VMEM((CHUNK, PDIM), jnp.float32),
            pltpu.VMEM((CHUNK, PDIM), jnp.float32),
            pltpu.SemaphoreType.DMA,
            pltpu.SemaphoreType.DMA,
        ],
    )
    def k(table_hbm, idx_hbm, out_hbm, idx_v, buf0, buf1, sem0, sem1):
        wid = lax.axis_index("s") * NUM_CORES + lax.axis_index("c")
        base = wid * per_w
        pltpu.sync_copy(idx_hbm.at[pl.ds(base, per_w)], idx_v)

        pltpu.async_copy(table_hbm.at[idx_v.at[pl.ds(0, CHUNK)]], buf0, sem0)
        pltpu.async_copy(table_hbm.at[idx_v.at[pl.ds(CHUNK, CHUNK)]], buf1, sem1)

        def body(j, carry):
            c0 = 2 * j
            pltpu.make_async_copy(
                table_hbm.at[idx_v.at[pl.ds(c0 * CHUNK, CHUNK)]], buf0, sem0
            ).wait()
            pltpu.sync_copy(buf0, out_hbm.at[pl.ds(base + c0 * CHUNK, CHUNK)])

            @pl.when(c0 + 2 < n_chunks)
            def _():
                pltpu.async_copy(
                    table_hbm.at[idx_v.at[pl.ds((c0 + 2) * CHUNK, CHUNK)]],
                    buf0,
                    sem0,
                )

            c1 = c0 + 1
            pltpu.make_async_copy(
                table_hbm.at[idx_v.at[pl.ds(c1 * CHUNK, CHUNK)]], buf1, sem1
            ).wait()
            pltpu.sync_copy(buf1, out_hbm.at[pl.ds(base + c1 * CHUNK, CHUNK)])

            @pl.when(c1 + 2 < n_chunks)
            def _():
                pltpu.async_copy(
                    table_hbm.at[idx_v.at[pl.ds((c1 + 2) * CHUNK, CHUNK)]],
                    buf1,
                    sem1,
                )

            return carry

        lax.fori_loop(0, n_chunks // 2, body, 0)

    return k(table_flat, idx_flat)


def kernel(inputs, tables):
    fields, vocab, dim = tables.shape
    batch = inputs.shape[0]
    offsets = (jnp.arange(fields, dtype=jnp.int32) * vocab)[None, :]
    idx_flat = (inputs + offsets).reshape(batch * fields)
    tpad = jnp.pad(tables, ((0, 0), (0, 0), (0, PDIM - dim)))
    table_flat = tpad.reshape(fields * vocab, PDIM)
    out = _sc_gather(table_flat, idx_flat)
    return out[:, :dim].reshape(batch, fields, dim)
